# baseline (speedup 1.0000x reference)
import functools

import jax
import jax.numpy as jnp
from jax import lax
from jax.experimental import pallas as pl
from jax.experimental.pallas import tpu as pltpu
from jax.experimental.pallas import tpu_sc as plsc

_B, _S, _D = 4, 2048, 1024
_NC, _NS, _L = 2, 16, 16
_NW = _NC * _NS
_B_SC = 1
_ROWS_W = _S // _NW
_CH = 16
_NQ = _ROWS_W // _CH
_NCHUNK = _B_SC * _NQ
_NBUF = 3


def _sc_body(x_hbm, t_hbm, o_hbm, t_buf, xb0, xb1, xb2,
             t_sem, in_sem0, in_sem1, in_sem2, out_sem0, out_sem1, out_sem2):
    wid = lax.axis_index("s") * _NC + lax.axis_index("c")
    s0 = wid * _ROWS_W

    bufs = (xb0, xb1, xb2)
    in_sems = (in_sem0, in_sem1, in_sem2)
    out_sems = (out_sem0, out_sem1, out_sem2)

    def chunk_slice(ref, i):
        b, q = divmod(i, _NQ)
        return ref.at[b, pl.ds(s0 + q * _CH, _CH), :]

    t_cp = pltpu.make_async_copy(
        t_hbm.at[pl.ds(s0, _ROWS_W), :], t_buf, t_sem)
    t_cp.start()
    in_cp = [None] * _NCHUNK
    out_cp = [None] * _NCHUNK
    in_cp[0] = pltpu.make_async_copy(chunk_slice(x_hbm, 0), bufs[0], in_sems[0])
    in_cp[0].start()
    t_cp.wait()

    for i in range(_NCHUNK):
        buf = bufs[i % _NBUF]
        in_cp[i].wait()
        if i + 1 < _NCHUNK:
            if i >= 2:
                out_cp[i - 2].wait()
            in_cp[i + 1] = pltpu.make_async_copy(
                chunk_slice(x_hbm, i + 1),
                bufs[(i + 1) % _NBUF], in_sems[(i + 1) % _NBUF])
            in_cp[i + 1].start()

        t_row0 = (i % _NQ) * _CH

        @plsc.parallel_loop(0, _CH * _D, _L, unroll=8)
        def _(j):
            r = lax.shift_right_logical(j, 10)
            c = pl.multiple_of(lax.bitwise_and(j, _D - 1), _L)
            v = t_buf[t_row0 + r, pl.ds(c, _L)]
            plsc.addupdate(buf.at[r, pl.ds(c, _L)], v)

        out_cp[i] = pltpu.make_async_copy(
            buf, chunk_slice(o_hbm, i), out_sems[i % _NBUF])
        out_cp[i].start()

    out_cp[_NCHUNK - 2].wait()
    out_cp[_NCHUNK - 1].wait()


_sc_kernel = functools.partial(
    pl.kernel,
    out_type=jax.ShapeDtypeStruct((_B_SC, _S, _D), jnp.float32),
    mesh=plsc.VectorSubcoreMesh(
        core_axis_name="c", subcore_axis_name="s",
        num_cores=_NC, num_subcores=_NS),
    scratch_types=[
        pltpu.VMEM((_ROWS_W, _D), jnp.float32),
        pltpu.VMEM((_CH, _D), jnp.float32),
        pltpu.VMEM((_CH, _D), jnp.float32),
        pltpu.VMEM((_CH, _D), jnp.float32),
        pltpu.SemaphoreType.DMA,
        pltpu.SemaphoreType.DMA,
        pltpu.SemaphoreType.DMA,
        pltpu.SemaphoreType.DMA,
        pltpu.SemaphoreType.DMA,
        pltpu.SemaphoreType.DMA,
        pltpu.SemaphoreType.DMA,
    ],
)(_sc_body)


def _tc_body(x_ref, t_ref, o_ref):
    o_ref[...] = x_ref[...] + t_ref[...]


_TC_BS = 256


def _tc_kernel(x, pos_table):
    return pl.pallas_call(
        _tc_body,
        grid=(_S // _TC_BS, _B - _B_SC),
        in_specs=[
            pl.BlockSpec((1, _TC_BS, _D), lambda s, b: (b + _B_SC, s, 0)),
            pl.BlockSpec((_TC_BS, _D), lambda s, b: (s, 0)),
        ],
        out_specs=pl.BlockSpec((1, _TC_BS, _D), lambda s, b: (b, s, 0)),
        out_shape=jax.ShapeDtypeStruct((_B - _B_SC, _S, _D), x.dtype),
    )(x, pos_table)


def kernel(x, pos_table):
    head = _sc_kernel(x, pos_table)
    tail = _tc_kernel(x, pos_table)
    return lax.concatenate([head, tail], 0)


# speedup vs baseline: 1.1210x; 1.1210x over previous
import functools

import jax
import jax.numpy as jnp
from jax import lax
from jax.experimental import pallas as pl
from jax.experimental.pallas import tpu as pltpu
from jax.experimental.pallas import tpu_sc as plsc

_B, _S, _D = 4, 2048, 1024
_NC, _NS, _L = 2, 16, 16
_NW = _NC * _NS
_ROWS_W = _S // _NW
_CH = 8
_NQ = _ROWS_W // _CH
_NCHUNK = _B * _NQ


_NBUF = 4


def _sc_body(x_hbm, t_hbm, o_hbm, t_buf, xb0, xb1, xb2, xb3,
             t_sem, in_sem0, in_sem1, in_sem2, in_sem3,
             out_sem0, out_sem1, out_sem2, out_sem3):
    wid = lax.axis_index("s") * _NC + lax.axis_index("c")
    s0 = wid * _ROWS_W

    bufs = (xb0, xb1, xb2, xb3)
    in_sems = (in_sem0, in_sem1, in_sem2, in_sem3)
    out_sems = (out_sem0, out_sem1, out_sem2, out_sem3)

    def chunk_slice(ref, i):
        b, q = divmod(i, _NQ)
        return ref.at[b, pl.ds(s0 + q * _CH, _CH), :]

    t_cp = pltpu.make_async_copy(
        t_hbm.at[pl.ds(s0, _ROWS_W), :], t_buf, t_sem)
    t_cp.start()
    in_cp = [None] * _NCHUNK
    out_cp = [None] * _NCHUNK
    in_cp[0] = pltpu.make_async_copy(chunk_slice(x_hbm, 0), bufs[0], in_sems[0])
    in_cp[0].start()
    t_cp.wait()

    for i in range(_NCHUNK):
        buf = bufs[i % _NBUF]
        in_cp[i].wait()
        if i + 1 < _NCHUNK:
            if i >= _NBUF - 1:
                out_cp[i - (_NBUF - 1)].wait()
            in_cp[i + 1] = pltpu.make_async_copy(
                chunk_slice(x_hbm, i + 1),
                bufs[(i + 1) % _NBUF], in_sems[(i + 1) % _NBUF])
            in_cp[i + 1].start()

        t_row0 = (i % _NQ) * _CH

        @plsc.parallel_loop(0, _CH * _D, _L, unroll=8)
        def _(j):
            r = lax.shift_right_logical(j, 10)
            c = pl.multiple_of(lax.bitwise_and(j, _D - 1), _L)
            v = t_buf[t_row0 + r, pl.ds(c, _L)]
            plsc.addupdate(buf.at[r, pl.ds(c, _L)], v)

        out_cp[i] = pltpu.make_async_copy(
            buf, chunk_slice(o_hbm, i), out_sems[i % _NBUF])
        out_cp[i].start()

    for k in range(_NBUF - 1):
        out_cp[_NCHUNK - 1 - k].wait()


_sc_kernel = functools.partial(
    pl.kernel,
    out_type=jax.ShapeDtypeStruct((_B, _S, _D), jnp.float32),
    mesh=plsc.VectorSubcoreMesh(
        core_axis_name="c", subcore_axis_name="s",
        num_cores=_NC, num_subcores=_NS),
    scratch_types=[
        pltpu.VMEM((_ROWS_W, _D), jnp.float32),
        pltpu.VMEM((_CH, _D), jnp.float32),
        pltpu.VMEM((_CH, _D), jnp.float32),
        pltpu.VMEM((_CH, _D), jnp.float32),
        pltpu.VMEM((_CH, _D), jnp.float32),
        pltpu.SemaphoreType.DMA,
        pltpu.SemaphoreType.DMA,
        pltpu.SemaphoreType.DMA,
        pltpu.SemaphoreType.DMA,
        pltpu.SemaphoreType.DMA,
        pltpu.SemaphoreType.DMA,
        pltpu.SemaphoreType.DMA,
        pltpu.SemaphoreType.DMA,
        pltpu.SemaphoreType.DMA,
    ],
)(_sc_body)


def kernel(x, pos_table):
    return _sc_kernel(x, pos_table)


# speedup vs baseline: 1.3745x; 1.2261x over previous
import functools

import jax
import jax.numpy as jnp
from jax import lax
from jax.experimental import pallas as pl
from jax.experimental.pallas import tpu as pltpu
from jax.experimental.pallas import tpu_sc as plsc

_B, _S, _D = 4, 2048, 1024
_NC, _NS, _L = 2, 16, 16
_NW = _NC * _NS
_ROWS_W = _S // _NW
_CH = 32
_NQ = _ROWS_W // _CH
_NCHUNK = _B * _NQ
_NBUF = 2


def _sc_body(x_hbm, t_hbm, o_hbm, t_buf, xb0, xb1,
             t_sem, in_sem0, in_sem1, out_sem0, out_sem1):
    wid = lax.axis_index("s") * _NC + lax.axis_index("c")
    s0 = wid * _ROWS_W

    bufs = (xb0, xb1)
    in_sems = (in_sem0, in_sem1)
    out_sems = (out_sem0, out_sem1)

    def chunk_slice(ref, i):
        q, b = divmod(i, _B)
        return ref.at[b, pl.ds(s0 + q * _CH, _CH), :]

    def t_copy(q):
        return pltpu.make_async_copy(
            t_hbm.at[pl.ds(s0 + q * _CH, _CH), :], t_buf, t_sem)

    t_cp = t_copy(0)
    t_cp.start()
    in_cp = [None] * _NCHUNK
    out_cp = [None] * _NCHUNK
    in_cp[0] = pltpu.make_async_copy(chunk_slice(x_hbm, 0), bufs[0], in_sems[0])
    in_cp[0].start()
    t_cp.wait()

    for i in range(_NCHUNK):
        buf = bufs[i % _NBUF]
        in_cp[i].wait()
        if i + 1 < _NCHUNK:
            if i >= _NBUF - 1:
                out_cp[i - (_NBUF - 1)].wait()
            in_cp[i + 1] = pltpu.make_async_copy(
                chunk_slice(x_hbm, i + 1),
                bufs[(i + 1) % _NBUF], in_sems[(i + 1) % _NBUF])
            in_cp[i + 1].start()

        @plsc.parallel_loop(0, _CH * _D, _L, unroll=8)
        def _(j):
            r = lax.shift_right_logical(j, 10)
            c = pl.multiple_of(lax.bitwise_and(j, _D - 1), _L)
            v = t_buf[r, pl.ds(c, _L)]
            plsc.addupdate(buf.at[r, pl.ds(c, _L)], v)

        out_cp[i] = pltpu.make_async_copy(
            buf, chunk_slice(o_hbm, i), out_sems[i % _NBUF])
        out_cp[i].start()

        if i % _B == _B - 1 and i + 1 < _NCHUNK:
            t_cp = t_copy((i + 1) // _B)
            t_cp.start()
            t_cp.wait()

    for k in range(_NBUF):
        out_cp[_NCHUNK - 1 - k].wait()


_sc_kernel = functools.partial(
    pl.kernel,
    out_type=jax.ShapeDtypeStruct((_B, _S, _D), jnp.float32),
    mesh=plsc.VectorSubcoreMesh(
        core_axis_name="c", subcore_axis_name="s",
        num_cores=_NC, num_subcores=_NS),
    scratch_types=[
        pltpu.VMEM((_CH, _D), jnp.float32),
        pltpu.VMEM((_CH, _D), jnp.float32),
        pltpu.VMEM((_CH, _D), jnp.float32),
        pltpu.SemaphoreType.DMA,
        pltpu.SemaphoreType.DMA,
        pltpu.SemaphoreType.DMA,
        pltpu.SemaphoreType.DMA,
        pltpu.SemaphoreType.DMA,
    ],
)(_sc_body)


def kernel(x, pos_table):
    return _sc_kernel(x, pos_table)


# speedup vs baseline: 1.3832x; 1.0063x over previous
import functools

import jax
import jax.numpy as jnp
from jax import lax
from jax.experimental import pallas as pl
from jax.experimental.pallas import tpu as pltpu
from jax.experimental.pallas import tpu_sc as plsc

_B, _S, _D = 4, 2048, 1024
_NC, _NS, _L = 2, 16, 16
_NW = _NC * _NS
_ROWS_W = _S // _NW
_CH = 16
_NQ = _ROWS_W // _CH
_NCHUNK = _B * _NQ


_NBUF = 3


def _sc_body(x_hbm, t_hbm, o_hbm, t_buf, xb0, xb1, xb2,
             t_sem, in_sem0, in_sem1, in_sem2, out_sem0, out_sem1, out_sem2):
    wid = lax.axis_index("s") * _NC + lax.axis_index("c")
    s0 = wid * _ROWS_W

    bufs = (xb0, xb1, xb2)
    in_sems = (in_sem0, in_sem1, in_sem2)
    out_sems = (out_sem0, out_sem1, out_sem2)

    def chunk_slice(ref, i):
        b, q = divmod(i, _NQ)
        return ref.at[b, pl.ds(s0 + q * _CH, _CH), :]

    t_cp = pltpu.make_async_copy(
        t_hbm.at[pl.ds(s0, _ROWS_W), :], t_buf, t_sem)
    t_cp.start()
    in_cp = [None] * _NCHUNK
    out_cp = [None] * _NCHUNK
    in_cp[0] = pltpu.make_async_copy(chunk_slice(x_hbm, 0), bufs[0], in_sems[0])
    in_cp[0].start()
    t_cp.wait()

    for i in range(_NCHUNK):
        buf = bufs[i % _NBUF]
        in_cp[i].wait()
        if i + 1 < _NCHUNK:
            if i >= 2:
                out_cp[i - 2].wait()
            in_cp[i + 1] = pltpu.make_async_copy(
                chunk_slice(x_hbm, i + 1),
                bufs[(i + 1) % _NBUF], in_sems[(i + 1) % _NBUF])
            in_cp[i + 1].start()

        t_row0 = (i % _NQ) * _CH

        @plsc.parallel_loop(0, _CH * _D, _L, unroll=8)
        def _(j):
            r = lax.shift_right_logical(j, 10)
            c = pl.multiple_of(lax.bitwise_and(j, _D - 1), _L)
            v = t_buf[t_row0 + r, pl.ds(c, _L)]
            plsc.addupdate(buf.at[r, pl.ds(c, _L)], v)

        out_cp[i] = pltpu.make_async_copy(
            buf, chunk_slice(o_hbm, i), out_sems[i % _NBUF])
        out_cp[i].start()

    out_cp[_NCHUNK - 2].wait()
    out_cp[_NCHUNK - 1].wait()


_sc_kernel = functools.partial(
    pl.kernel,
    out_type=jax.ShapeDtypeStruct((_B, _S, _D), jnp.float32),
    mesh=plsc.VectorSubcoreMesh(
        core_axis_name="c", subcore_axis_name="s",
        num_cores=_NC, num_subcores=_NS),
    scratch_types=[
        pltpu.VMEM((_ROWS_W, _D), jnp.float32),
        pltpu.VMEM((_CH, _D), jnp.float32),
        pltpu.VMEM((_CH, _D), jnp.float32),
        pltpu.VMEM((_CH, _D), jnp.float32),
        pltpu.SemaphoreType.DMA,
        pltpu.SemaphoreType.DMA,
        pltpu.SemaphoreType.DMA,
        pltpu.SemaphoreType.DMA,
        pltpu.SemaphoreType.DMA,
        pltpu.SemaphoreType.DMA,
        pltpu.SemaphoreType.DMA,
    ],
)(_sc_body)


def kernel(x, pos_table):
    return _sc_kernel(x, pos_table)
